# baseline (speedup 1.0000x reference)
import jax
import jax.numpy as jnp
from jax import lax
from jax.experimental import pallas as pl
from jax.experimental.pallas import tpu as pltpu

_BN = 2048
_K = 6


def _fhs_body(surf_ref, emb_ref, o_ref):
    s = surf_ref[...]
    B, T = s.shape
    V = emb_ref.shape[0]
    oh = (s[:, :, None] == lax.broadcasted_iota(jnp.int32, (B, T, V), 2))
    counts = jnp.sum(oh.astype(jnp.float32), axis=1)
    o_ref[...] = jnp.dot(
        counts, emb_ref[...], preferred_element_type=jnp.float32) * (1.0 / T)


def _w_copy(w_hbm, wbuf, sems, blk, slot):
    return pltpu.make_async_copy(
        w_hbm.at[:, pl.ds(blk * _BN, _BN)], wbuf.at[slot], sems.at[slot])


def _proj_body(fhs_ref, b_ref, wtail_ref, w_hbm, o_ref, wbuf, sems):
    i = pl.program_id(0)
    nb = pl.num_programs(0)
    nfull = nb - 1

    @pl.when(i == 0)
    def _prologue():
        for k in range(_K):
            _w_copy(w_hbm, wbuf, sems, k, k).start()

    slot = lax.rem(i, _K)

    @pl.when(i < nfull)
    def _main():
        _w_copy(w_hbm, wbuf, sems, i, slot).wait()
        o_ref[...] = (
            jnp.dot(fhs_ref[...], wbuf[slot],
                    preferred_element_type=jnp.float32)
            + b_ref[...])

    @pl.when(i == nfull)
    def _tail():
        o_ref[...] = (
            jnp.dot(fhs_ref[...], wtail_ref[...],
                    preferred_element_type=jnp.float32)
            + b_ref[...])

    @pl.when(i + _K < nfull)
    def _refill():
        _w_copy(w_hbm, wbuf, sems, i + _K, slot).start()


def kernel(surf, char_emb, root_codebook, suffix_W, suffix_b, suffix_codebook,
           ord_W, ord_b, ord_codebooks, out_W, out_b):
    B, T = surf.shape
    V, D = char_emb.shape
    _, N = out_W.shape
    nb = (N + _BN - 1) // _BN
    b2d = out_b.reshape(1, N)

    fhs = pl.pallas_call(
        _fhs_body,
        out_shape=jax.ShapeDtypeStruct((B, D), jnp.float32),
    )(surf, char_emb)

    out2d = pl.pallas_call(
        _proj_body,
        grid=(nb,),
        in_specs=[
            pl.BlockSpec((B, D), lambda i: (0, 0)),
            pl.BlockSpec((1, _BN), lambda i: (0, i)),
            pl.BlockSpec((D, _BN), lambda i: (0, nb - 1)),
            pl.BlockSpec(memory_space=pl.ANY),
        ],
        out_specs=pl.BlockSpec((B, _BN), lambda i: (0, i)),
        out_shape=jax.ShapeDtypeStruct((B, N), jnp.float32),
        scratch_shapes=[
            pltpu.VMEM((_K, D, _BN), jnp.float32),
            pltpu.SemaphoreType.DMA((_K,)),
        ],
        compiler_params=pltpu.CompilerParams(
            dimension_semantics=("arbitrary",)),
    )(fhs, b2d, out_W, out_W)
    return out2d[:, None, :]
